# Initial kernel scaffold; baseline (speedup 1.0000x reference)
import functools

import jax
import jax.numpy as jnp
from jax import lax
from jax.experimental import pallas as pl
from jax.experimental.pallas import tpu as pltpu
from jax.experimental.pallas import tpu_sc as plsc

E = 8
TOP_K = 2
T = 4096
D = 2048
CAPACITY = 2048

NC = 2
NS = 16
L = 16
NW = NC * NS

N_OUT = E * CAPACITY
ROWS_PER_TILE = N_OUT // NW
CHUNK = 16
N_CHUNKS = ROWS_PER_TILE // CHUNK

_mesh = lambda: plsc.VectorSubcoreMesh(core_axis_name="c", subcore_axis_name="s")


def _worker_id():
    return lax.axis_index("s") * NC + lax.axis_index("c")


def _splat_i32(x):
    return jnp.zeros((L,), jnp.int32) + x


@functools.partial(
    pl.kernel,
    out_type=(
        jax.ShapeDtypeStruct((N_OUT // L, L), jnp.int32),
        jax.ShapeDtypeStruct((N_OUT // L, L), jnp.float32),
        jax.ShapeDtypeStruct((E, T), jnp.int32),
        jax.ShapeDtypeStruct((E, L), jnp.int32),
    ),
    mesh=_mesh(),
    scratch_types=[
        pltpu.VMEM((T,), jnp.int32),
        pltpu.VMEM((T,), jnp.float32),
        pltpu.VMEM((T,), jnp.int32),
        pltpu.VMEM((CAPACITY // L, L), jnp.int32),
        pltpu.VMEM((CAPACITY // L, L), jnp.float32),
        pltpu.VMEM((L,), jnp.int32),
    ],
)
def _index_kernel(hmT, scoreT, src_o, gate_o, lidx_o, loads_o,
                  hm_v, sc_v, li_v, src_v, gate_v, ld_v):
    wid = _worker_id()

    @pl.when(wid < E)
    def _():
        e = wid
        pltpu.sync_copy(hmT.at[e], hm_v)
        pltpu.sync_copy(scoreT.at[e], sc_v)

        lane = lax.iota(jnp.int32, L)

        def init_body(j, _):
            row = e * CAPACITY + j * L
            src_v[j] = (_splat_i32(row) + lane) & (T - 1)
            gate_v[j] = jnp.zeros((L,), jnp.float32)
            return 0

        lax.fori_loop(0, CAPACITY // L, init_body, 0, unroll=4)

        def body(i, carry):
            hm = hm_v[pl.ds(i * L, L)]
            cs = plsc.cumsum(hm)
            pos = cs - 1 + carry
            hot = hm > 0
            valid = hot & (pos < CAPACITY)
            tok = _splat_i32(i * L) + lane
            li_v[pl.ds(i * L, L)] = jnp.where(hot, pos, -1)
            plsc.store_scatter(src_v, [pos >> 4, pos & (L - 1)], tok, mask=valid)
            sc = sc_v[pl.ds(i * L, L)]
            plsc.store_scatter(gate_v, [pos >> 4, pos & (L - 1)], sc, mask=valid)
            return carry + jnp.max(cs)

        total = lax.fori_loop(0, T // L, body, jnp.int32(0), unroll=2)

        ld_v[...] = _splat_i32(total)
        rows = CAPACITY // L
        pltpu.sync_copy(src_v, src_o.at[pl.ds(e * rows, rows)])
        pltpu.sync_copy(gate_v, gate_o.at[pl.ds(e * rows, rows)])
        pltpu.sync_copy(li_v, lidx_o.at[e])
        pltpu.sync_copy(ld_v, loads_o.at[e])


@functools.partial(
    pl.kernel,
    out_type=jax.ShapeDtypeStruct((N_OUT, D), jnp.float32),
    mesh=_mesh(),
    scratch_types=[
        pltpu.VMEM((ROWS_PER_TILE // L, L), jnp.int32),
        pltpu.VMEM((ROWS_PER_TILE // L, L), jnp.float32),
        pltpu.VMEM((CHUNK, D), jnp.float32),
        pltpu.SemaphoreType.DMA,
    ],
)
def _dispatch_kernel(flow, src, gate, out, idx_v, gate_v, buf, sem):
    wid = _worker_id()
    vrows = ROWS_PER_TILE // L
    pltpu.sync_copy(src.at[pl.ds(wid * vrows, vrows)], idx_v)
    pltpu.sync_copy(gate.at[pl.ds(wid * vrows, vrows)], gate_v)
    base = wid * ROWS_PER_TILE

    def chunk_body(g, _):
        pltpu.async_copy(flow.at[idx_v.at[g]], buf, sem).wait()

        def row_body(r, _):
            gs = plsc.load_gather(gate_v, [_splat_i32(g), _splat_i32(r)])

            def col_body(d, _):
                buf[r, pl.ds(d * L, L)] = buf[r, pl.ds(d * L, L)] * gs
                return 0

            lax.fori_loop(0, D // L, col_body, 0, unroll=8)
            return 0

        lax.fori_loop(0, CHUNK, row_body, 0)
        pltpu.sync_copy(buf, out.at[pl.ds(base + g * CHUNK, CHUNK)])
        return 0

    lax.fori_loop(0, N_CHUNKS, chunk_body, 0)


def kernel(in_flow, hot_mask, score):
    hmT = jnp.transpose(hot_mask.astype(jnp.int32))
    scoreT = jnp.transpose(score)
    src2d, gate2d, lidxT, loads2d = _index_kernel(hmT, scoreT)
    out = _dispatch_kernel(in_flow, src2d, gate2d)
    return out, jnp.transpose(lidxT), loads2d[:, 0]


# speedup vs baseline: 9.5401x; 9.5401x over previous
import functools

import jax
import jax.numpy as jnp
from jax import lax
from jax.experimental import pallas as pl
from jax.experimental.pallas import tpu as pltpu
from jax.experimental.pallas import tpu_sc as plsc

E = 8
TOP_K = 2
T = 4096
D = 2048
CAPACITY = 2048

NC = 2
NS = 16
L = 16
NW = NC * NS

N_OUT = E * CAPACITY
ROWS_PER_TILE = N_OUT // NW
CHUNK = 16
N_CHUNKS = ROWS_PER_TILE // CHUNK

_mesh = lambda: plsc.VectorSubcoreMesh(core_axis_name="c", subcore_axis_name="s")


def _worker_id():
    return lax.axis_index("s") * NC + lax.axis_index("c")


def _splat_i32(x):
    return jnp.zeros((L,), jnp.int32) + x


@functools.partial(
    pl.kernel,
    out_type=(
        jax.ShapeDtypeStruct((N_OUT // L, L), jnp.int32),
        jax.ShapeDtypeStruct((N_OUT // L, L), jnp.float32),
        jax.ShapeDtypeStruct((E, T), jnp.int32),
        jax.ShapeDtypeStruct((E, L), jnp.int32),
    ),
    mesh=_mesh(),
    compiler_params=pltpu.CompilerParams(needs_layout_passes=False),
    scratch_types=[
        pltpu.VMEM((T,), jnp.int32),
        pltpu.VMEM((T,), jnp.float32),
        pltpu.VMEM((T,), jnp.int32),
        pltpu.VMEM((CAPACITY // L, L), jnp.int32),
        pltpu.VMEM((CAPACITY // L, L), jnp.float32),
        pltpu.VMEM((L,), jnp.int32),
    ],
)
def _index_kernel(hmT, scoreT, src_o, gate_o, lidx_o, loads_o,
                  hm_v, sc_v, li_v, src_v, gate_v, ld_v):
    wid = _worker_id()

    @pl.when(wid < E)
    def _():
        e = wid
        pltpu.sync_copy(hmT.at[e], hm_v)
        pltpu.sync_copy(scoreT.at[e], sc_v)

        lane = lax.iota(jnp.int32, L)

        def init_body(j, _):
            row = e * CAPACITY + j * L
            src_v[j] = (_splat_i32(row) + lane) & (T - 1)
            gate_v[j] = jnp.zeros((L,), jnp.float32)
            return 0

        lax.fori_loop(0, CAPACITY // L, init_body, 0, unroll=4)

        def body(i, carry):
            hm = hm_v[pl.ds(i * L, L)]
            cs = plsc.cumsum(hm)
            pos = cs - 1 + carry
            hot = hm > 0
            valid = hot & (pos < CAPACITY)
            tok = _splat_i32(i * L) + lane
            li_v[pl.ds(i * L, L)] = jnp.where(hot, pos, -1)
            plsc.store_scatter(src_v, [pos >> 4, pos & (L - 1)], tok, mask=valid)
            sc = sc_v[pl.ds(i * L, L)]
            plsc.store_scatter(gate_v, [pos >> 4, pos & (L - 1)], sc, mask=valid)
            return carry + jnp.max(cs)

        total = lax.fori_loop(0, T // L, body, jnp.int32(0), unroll=2)

        ld_v[...] = _splat_i32(total)
        rows = CAPACITY // L
        pltpu.sync_copy(src_v, src_o.at[pl.ds(e * rows, rows)])
        pltpu.sync_copy(gate_v, gate_o.at[pl.ds(e * rows, rows)])
        pltpu.sync_copy(li_v, lidx_o.at[e])
        pltpu.sync_copy(ld_v, loads_o.at[e])


@functools.partial(
    pl.kernel,
    out_type=jax.ShapeDtypeStruct((N_OUT, D), jnp.float32),
    mesh=_mesh(),
    compiler_params=pltpu.CompilerParams(needs_layout_passes=False),
    scratch_types=[
        pltpu.VMEM((ROWS_PER_TILE // L, L), jnp.int32),
        pltpu.VMEM((ROWS_PER_TILE // L, L), jnp.float32),
        pltpu.VMEM((CHUNK, D), jnp.float32),
        pltpu.SemaphoreType.DMA,
    ],
)
def _dispatch_kernel(flow, src, gate, out, idx_v, gate_v, buf, sem):
    wid = _worker_id()
    vrows = ROWS_PER_TILE // L
    pltpu.sync_copy(src.at[pl.ds(wid * vrows, vrows)], idx_v)
    pltpu.sync_copy(gate.at[pl.ds(wid * vrows, vrows)], gate_v)
    base = wid * ROWS_PER_TILE

    def chunk_body(g, _):
        pltpu.async_copy(flow.at[idx_v.at[g]], buf, sem).wait()

        def row_body(r, _):
            gs = plsc.load_gather(gate_v, [_splat_i32(g), _splat_i32(r)])

            def col_body(d, _):
                buf[r, pl.ds(d * L, L)] = buf[r, pl.ds(d * L, L)] * gs
                return 0

            lax.fori_loop(0, D // L, col_body, 0, unroll=8)
            return 0

        lax.fori_loop(0, CHUNK, row_body, 0)
        pltpu.sync_copy(buf, out.at[pl.ds(base + g * CHUNK, CHUNK)])
        return 0

    lax.fori_loop(0, N_CHUNKS, chunk_body, 0)


def kernel(in_flow, hot_mask, score):
    hmT = jnp.transpose(hot_mask.astype(jnp.int32))
    scoreT = jnp.transpose(score)
    src2d, gate2d, lidxT, loads2d = _index_kernel(hmT, scoreT)
    out = _dispatch_kernel(in_flow, src2d, gate2d)
    return out, jnp.transpose(lidxT), loads2d[:, 0]


# speedup vs baseline: 13.4788x; 1.4129x over previous
import functools

import jax
import jax.numpy as jnp
from jax import lax
from jax.experimental import pallas as pl
from jax.experimental.pallas import tpu as pltpu
from jax.experimental.pallas import tpu_sc as plsc

E = 8
TOP_K = 2
T = 4096
D = 2048
CAPACITY = 2048

NC = 2
NS = 16
L = 16
NW = NC * NS

N_OUT = E * CAPACITY
ROWS_PER_TILE = N_OUT // NW
CHUNK = 16
N_CHUNKS = ROWS_PER_TILE // CHUNK

_mesh = lambda: plsc.VectorSubcoreMesh(core_axis_name="c", subcore_axis_name="s")


def _worker_id():
    return lax.axis_index("s") * NC + lax.axis_index("c")


def _splat_i32(x):
    return jnp.zeros((L,), jnp.int32) + x


@functools.partial(
    pl.kernel,
    out_type=(
        jax.ShapeDtypeStruct((N_OUT // L, L), jnp.int32),
        jax.ShapeDtypeStruct((N_OUT // L, L), jnp.float32),
        jax.ShapeDtypeStruct((E, T), jnp.int32),
        jax.ShapeDtypeStruct((E, L), jnp.int32),
    ),
    mesh=_mesh(),
    compiler_params=pltpu.CompilerParams(needs_layout_passes=False),
    scratch_types=[
        pltpu.VMEM((T,), jnp.int32),
        pltpu.VMEM((T,), jnp.float32),
        pltpu.VMEM((T,), jnp.int32),
        pltpu.VMEM((CAPACITY // L, L), jnp.int32),
        pltpu.VMEM((CAPACITY // L, L), jnp.float32),
        pltpu.VMEM((L,), jnp.int32),
    ],
)
def _index_kernel(hmT, scoreT, src_o, gate_o, lidx_o, loads_o,
                  hm_v, sc_v, li_v, src_v, gate_v, ld_v):
    wid = _worker_id()

    @pl.when(wid < E)
    def _():
        e = wid
        pltpu.sync_copy(hmT.at[e], hm_v)
        pltpu.sync_copy(scoreT.at[e], sc_v)

        lane = lax.iota(jnp.int32, L)

        def init_body(j, _):
            row = e * CAPACITY + j * L
            src_v[j] = (_splat_i32(row) + lane) & (T - 1)
            gate_v[j] = jnp.zeros((L,), jnp.float32)
            return 0

        lax.fori_loop(0, CAPACITY // L, init_body, 0, unroll=4)

        def body(i, carry):
            hm = hm_v[pl.ds(i * L, L)]
            cs = plsc.cumsum(hm)
            pos = cs - 1 + carry
            hot = hm > 0
            valid = hot & (pos < CAPACITY)
            tok = _splat_i32(i * L) + lane
            li_v[pl.ds(i * L, L)] = jnp.where(hot, pos, -1)
            plsc.store_scatter(src_v, [pos >> 4, pos & (L - 1)], tok, mask=valid)
            sc = sc_v[pl.ds(i * L, L)]
            plsc.store_scatter(gate_v, [pos >> 4, pos & (L - 1)], sc, mask=valid)
            return carry + jnp.max(cs)

        total = lax.fori_loop(0, T // L, body, jnp.int32(0), unroll=2)

        ld_v[...] = _splat_i32(total)
        rows = CAPACITY // L
        pltpu.sync_copy(src_v, src_o.at[pl.ds(e * rows, rows)])
        pltpu.sync_copy(gate_v, gate_o.at[pl.ds(e * rows, rows)])
        pltpu.sync_copy(li_v, lidx_o.at[e])
        pltpu.sync_copy(ld_v, loads_o.at[e])


TILES_PER_EXPERT = NW // E


@functools.partial(
    pl.kernel,
    out_type=jax.ShapeDtypeStruct((N_OUT, D), jnp.float32),
    mesh=_mesh(),
    compiler_params=pltpu.CompilerParams(needs_layout_passes=False),
    scratch_types=[
        pltpu.VMEM((ROWS_PER_TILE // L, L), jnp.int32),
        pltpu.VMEM((ROWS_PER_TILE // L, L), jnp.float32),
        pltpu.VMEM((L,), jnp.int32),
        pltpu.VMEM((2, CHUNK, D), jnp.float32),
        pltpu.VMEM((CHUNK, D), jnp.float32),
        pltpu.SemaphoreType.DMA,
        pltpu.SemaphoreType.DMA,
        pltpu.SemaphoreType.DMA,
        pltpu.SemaphoreType.DMA,
        pltpu.SemaphoreType.DMA,
    ],
)
def _dispatch_kernel(flow, src, gate, loads, zeros, out,
                     idx_v, gate_v, ld_v, bufs, zbuf, sg0, sg1, ss0, ss1, sz):
    wid = _worker_id()
    vrows = ROWS_PER_TILE // L
    pltpu.sync_copy(src.at[pl.ds(wid * vrows, vrows)], idx_v)
    pltpu.sync_copy(gate.at[pl.ds(wid * vrows, vrows)], gate_v)
    pltpu.sync_copy(loads.at[wid // TILES_PER_EXPERT], ld_v)
    pltpu.sync_copy(zeros, zbuf)
    base = wid * ROWS_PER_TILE

    ld = jnp.max(ld_v[...])
    nv = jnp.clip(ld - (wid % TILES_PER_EXPERT) * ROWS_PER_TILE, 0, ROWS_PER_TILE)
    ngc = (nv + CHUNK - 1) // CHUNK

    sgs = (sg0, sg1)
    sss = (ss0, ss1)

    def out_rows(g):
        return out.at[pl.ds(base + g * CHUNK, CHUNK)]

    def gather_issue(g):
        @pl.when(g < ngc)
        def _():
            pltpu.async_copy(flow.at[idx_v.at[g]], bufs.at[g % 2], sgs[g % 2])

    def gather_wait(g):
        pltpu.make_async_copy(flow.at[idx_v.at[g]], bufs.at[g % 2], sgs[g % 2]).wait()

    def store_wait(g):
        @pl.when(g < ngc)
        def _():
            pltpu.make_async_copy(bufs.at[g % 2], out_rows(g), sss[g % 2]).wait()

    gather_issue(0)
    for g in range(N_CHUNKS):
        b = g % 2
        if g + 1 < N_CHUNKS:
            if g >= 1:
                store_wait(g - 1)
            gather_issue(g + 1)

        @pl.when(g < ngc)
        def _():
            gather_wait(g)

            def row_body(r, _):
                gs = plsc.load_gather(gate_v, [_splat_i32(g), _splat_i32(r)])

                def col_body(d, _):
                    bufs[b, r, pl.ds(d * L, L)] = bufs[b, r, pl.ds(d * L, L)] * gs
                    return 0

                lax.fori_loop(0, D // L, col_body, 0, unroll=8)
                return 0

            lax.fori_loop(0, CHUNK, row_body, 0)
            pltpu.async_copy(bufs.at[b], out_rows(g), sss[b])

        @pl.when(g >= ngc)
        def _():
            pltpu.async_copy(zbuf, out_rows(g), sz)

    store_wait(N_CHUNKS - 1)

    def zdrain(i, _):
        pltpu.make_async_copy(zbuf, out_rows(0), sz).wait()
        return 0

    lax.fori_loop(ngc, N_CHUNKS, zdrain, 0)


def kernel(in_flow, hot_mask, score):
    hmT = jnp.transpose(hot_mask.astype(jnp.int32))
    scoreT = jnp.transpose(score)
    src2d, gate2d, lidxT, loads2d = _index_kernel(hmT, scoreT)
    zeros = jnp.zeros((CHUNK, D), jnp.float32)
    out = _dispatch_kernel(in_flow, src2d, gate2d, loads2d, zeros)
    return out, jnp.transpose(lidxT), loads2d[:, 0]


# speedup vs baseline: 15.7577x; 1.1691x over previous
import functools

import jax
import jax.numpy as jnp
from jax import lax
from jax.experimental import pallas as pl
from jax.experimental.pallas import tpu as pltpu
from jax.experimental.pallas import tpu_sc as plsc

E = 8
TOP_K = 2
T = 4096
D = 2048
CAPACITY = 2048

NC = 2
NS = 16
L = 16
NW = NC * NS

N_OUT = E * CAPACITY
ROWS_PER_TILE = N_OUT // NW
CHUNK = 16
N_CHUNKS = ROWS_PER_TILE // CHUNK

_mesh = lambda: plsc.VectorSubcoreMesh(core_axis_name="c", subcore_axis_name="s")


def _worker_id():
    return lax.axis_index("s") * NC + lax.axis_index("c")


def _splat_i32(x):
    return jnp.zeros((L,), jnp.int32) + x


@functools.partial(
    pl.kernel,
    out_type=(
        jax.ShapeDtypeStruct((N_OUT // L, L), jnp.int32),
        jax.ShapeDtypeStruct((N_OUT // L, L), jnp.float32),
        jax.ShapeDtypeStruct((E, T), jnp.int32),
        jax.ShapeDtypeStruct((E, L), jnp.int32),
    ),
    mesh=_mesh(),
    compiler_params=pltpu.CompilerParams(needs_layout_passes=False),
    scratch_types=[
        pltpu.VMEM((T,), jnp.int32),
        pltpu.VMEM((T,), jnp.float32),
        pltpu.VMEM((T,), jnp.int32),
        pltpu.VMEM((CAPACITY // L, L), jnp.int32),
        pltpu.VMEM((CAPACITY // L, L), jnp.float32),
        pltpu.VMEM((L,), jnp.int32),
    ],
)
def _index_kernel(hmT, scoreT, src_o, gate_o, lidx_o, loads_o,
                  hm_v, sc_v, li_v, src_v, gate_v, ld_v):
    wid = _worker_id()

    @pl.when(wid < E)
    def _():
        e = wid
        pltpu.sync_copy(hmT.at[e], hm_v)
        pltpu.sync_copy(scoreT.at[e], sc_v)

        lane = lax.iota(jnp.int32, L)

        def init_body(j, _):
            row = e * CAPACITY + j * L
            src_v[j] = (_splat_i32(row) + lane) & (T - 1)
            gate_v[j] = jnp.zeros((L,), jnp.float32)
            return 0

        lax.fori_loop(0, CAPACITY // L, init_body, 0, unroll=4)

        def body(i, carry):
            hm = hm_v[pl.ds(i * L, L)]
            cs = plsc.cumsum(hm)
            pos = cs - 1 + carry
            hot = hm > 0
            valid = hot & (pos < CAPACITY)
            tok = _splat_i32(i * L) + lane
            li_v[pl.ds(i * L, L)] = jnp.where(hot, pos, -1)
            row = pos >> 4
            prow = (row & 3) * 32 + (row >> 2)
            plsc.store_scatter(src_v, [prow, pos & (L - 1)], tok, mask=valid)
            sc = sc_v[pl.ds(i * L, L)]
            plsc.store_scatter(gate_v, [prow, pos & (L - 1)], sc, mask=valid)
            return carry + jnp.max(cs)

        total = lax.fori_loop(0, T // L, body, jnp.int32(0), unroll=2)

        ld_v[...] = _splat_i32(total)
        rows = CAPACITY // L
        pltpu.sync_copy(src_v, src_o.at[pl.ds(e * rows, rows)])
        pltpu.sync_copy(gate_v, gate_o.at[pl.ds(e * rows, rows)])
        pltpu.sync_copy(li_v, lidx_o.at[e])
        pltpu.sync_copy(ld_v, loads_o.at[e])


TILES_PER_EXPERT = NW // E


@functools.partial(
    pl.kernel,
    out_type=jax.ShapeDtypeStruct((N_OUT, D), jnp.float32),
    mesh=_mesh(),
    compiler_params=pltpu.CompilerParams(needs_layout_passes=False),
    scratch_types=[
        pltpu.VMEM((ROWS_PER_TILE // L, L), jnp.int32),
        pltpu.VMEM((ROWS_PER_TILE // L, L), jnp.float32),
        pltpu.VMEM((L,), jnp.int32),
        pltpu.VMEM((2, CHUNK, D), jnp.float32),
        pltpu.VMEM((CHUNK, D), jnp.float32),
        pltpu.SemaphoreType.DMA,
        pltpu.SemaphoreType.DMA,
        pltpu.SemaphoreType.DMA,
        pltpu.SemaphoreType.DMA,
        pltpu.SemaphoreType.DMA,
    ],
)
def _dispatch_kernel(flow, src, gate, loads, zeros, out,
                     idx_v, gate_v, ld_v, bufs, zbuf, sg0, sg1, ss0, ss1, sz):
    wid = _worker_id()
    e = wid // TILES_PER_EXPERT
    j = wid % TILES_PER_EXPERT
    vrows = ROWS_PER_TILE // L
    pltpu.sync_copy(src.at[pl.ds(wid * vrows, vrows)], idx_v)
    pltpu.sync_copy(gate.at[pl.ds(wid * vrows, vrows)], gate_v)
    pltpu.sync_copy(loads.at[e], ld_v)
    pltpu.sync_copy(zeros, zbuf)

    ld = jnp.max(ld_v[...])
    nc_expert = jnp.clip((ld + CHUNK - 1) // CHUNK, 0, CAPACITY // CHUNK)
    ngc = jnp.clip((nc_expert - j + TILES_PER_EXPERT - 1) // TILES_PER_EXPERT,
                   0, N_CHUNKS)

    sgs = (sg0, sg1)
    sss = (ss0, ss1)
    obase = e * CAPACITY + j * CHUNK

    def out_rows(g):
        return out.at[pl.ds(obase + g * (TILES_PER_EXPERT * CHUNK), CHUNK)]

    def gather_issue(g):
        @pl.when(g < ngc)
        def _():
            pltpu.async_copy(flow.at[idx_v.at[g]], bufs.at[g % 2], sgs[g % 2])

    def gather_wait(g):
        pltpu.make_async_copy(flow.at[idx_v.at[g]], bufs.at[g % 2], sgs[g % 2]).wait()

    def store_wait(g):
        @pl.when(g < ngc)
        def _():
            pltpu.make_async_copy(bufs.at[g % 2], out_rows(g), sss[g % 2]).wait()

    gather_issue(0)
    for g in range(N_CHUNKS):
        b = g % 2
        if g + 1 < N_CHUNKS:
            if g >= 1:
                store_wait(g - 1)
            gather_issue(g + 1)

        @pl.when(g < ngc)
        def _():
            gather_wait(g)

            def row_body(r, _):
                gs = plsc.load_gather(gate_v, [_splat_i32(g), _splat_i32(r)])

                def col_body(d, _):
                    bufs[b, r, pl.ds(d * L, L)] = bufs[b, r, pl.ds(d * L, L)] * gs
                    return 0

                lax.fori_loop(0, D // L, col_body, 0, unroll=8)
                return 0

            lax.fori_loop(0, CHUNK, row_body, 0)
            pltpu.async_copy(bufs.at[b], out_rows(g), sss[b])

        @pl.when(g >= ngc)
        def _():
            pltpu.async_copy(zbuf, out_rows(g), sz)

    store_wait(N_CHUNKS - 1)

    def zdrain(i, _):
        pltpu.make_async_copy(zbuf, out_rows(0), sz).wait()
        return 0

    lax.fori_loop(ngc, N_CHUNKS, zdrain, 0)


def kernel(in_flow, hot_mask, score):
    hmT = jnp.transpose(hot_mask.astype(jnp.int32))
    scoreT = jnp.transpose(score)
    src2d, gate2d, lidxT, loads2d = _index_kernel(hmT, scoreT)
    zeros = jnp.zeros((CHUNK, D), jnp.float32)
    out = _dispatch_kernel(in_flow, src2d, gate2d, loads2d, zeros)
    return out, jnp.transpose(lidxT), loads2d[:, 0]


# speedup vs baseline: 15.7906x; 1.0021x over previous
import functools

import jax
import jax.numpy as jnp
from jax import lax
from jax.experimental import pallas as pl
from jax.experimental.pallas import tpu as pltpu
from jax.experimental.pallas import tpu_sc as plsc

E = 8
TOP_K = 2
T = 4096
D = 2048
CAPACITY = 2048

NC = 2
NS = 16
L = 16
NW = NC * NS

N_OUT = E * CAPACITY
ROWS_PER_TILE = N_OUT // NW
CHUNK = 16
N_CHUNKS = ROWS_PER_TILE // CHUNK

_mesh = lambda: plsc.VectorSubcoreMesh(core_axis_name="c", subcore_axis_name="s")


def _worker_id():
    return lax.axis_index("s") * NC + lax.axis_index("c")


def _splat_i32(x):
    return jnp.zeros((L,), jnp.int32) + x


@functools.partial(
    pl.kernel,
    out_type=(
        jax.ShapeDtypeStruct((N_OUT // L, L), jnp.int32),
        jax.ShapeDtypeStruct((N_OUT // L, L), jnp.float32),
        jax.ShapeDtypeStruct((E, T), jnp.int32),
        jax.ShapeDtypeStruct((E, L), jnp.int32),
    ),
    mesh=_mesh(),
    compiler_params=pltpu.CompilerParams(needs_layout_passes=False),
    scratch_types=[
        pltpu.VMEM((T,), jnp.int32),
        pltpu.VMEM((T,), jnp.float32),
        pltpu.VMEM((T,), jnp.int32),
        pltpu.VMEM((CAPACITY // L, L), jnp.int32),
        pltpu.VMEM((CAPACITY // L, L), jnp.float32),
        pltpu.VMEM((L,), jnp.int32),
    ],
)
def _index_kernel(hmT, scoreT, src_o, gate_o, lidx_o, loads_o,
                  hm_v, sc_v, li_v, src_v, gate_v, ld_v):
    wid = _worker_id()

    @pl.when(wid < E)
    def _():
        e = wid
        pltpu.sync_copy(hmT.at[e], hm_v)
        pltpu.sync_copy(scoreT.at[e], sc_v)

        lane = lax.iota(jnp.int32, L)

        def init_body(j, _):
            row = e * CAPACITY + j * L
            src_v[j] = (_splat_i32(row) + lane) & (T - 1)
            gate_v[j] = jnp.zeros((L,), jnp.float32)
            return 0

        lax.fori_loop(0, CAPACITY // L, init_body, 0, unroll=4)

        def body(i, carry):
            hm = hm_v[pl.ds(i * L, L)]
            cs = plsc.cumsum(hm)
            pos = cs - 1 + carry
            hot = hm > 0
            valid = hot & (pos < CAPACITY)
            tok = _splat_i32(i * L) + lane
            li_v[pl.ds(i * L, L)] = jnp.where(hot, pos, -1)
            row = pos >> 4
            prow = (row & 3) * 32 + (row >> 2)
            plsc.store_scatter(src_v, [prow, pos & (L - 1)], tok, mask=valid)
            sc = sc_v[pl.ds(i * L, L)]
            plsc.store_scatter(gate_v, [prow, pos & (L - 1)], sc, mask=valid)
            return carry + jnp.max(cs)

        total = lax.fori_loop(0, T // L, body, jnp.int32(0), unroll=2)

        ld_v[...] = _splat_i32(total)
        rows = CAPACITY // L
        pltpu.sync_copy(src_v, src_o.at[pl.ds(e * rows, rows)])
        pltpu.sync_copy(gate_v, gate_o.at[pl.ds(e * rows, rows)])
        pltpu.sync_copy(li_v, lidx_o.at[e])
        pltpu.sync_copy(ld_v, loads_o.at[e])


TILES_PER_EXPERT = NW // E


@functools.partial(
    pl.kernel,
    out_type=jax.ShapeDtypeStruct((N_OUT, D), jnp.float32),
    mesh=_mesh(),
    compiler_params=pltpu.CompilerParams(needs_layout_passes=False),
    scratch_types=[
        pltpu.VMEM((ROWS_PER_TILE // L, L), jnp.int32),
        pltpu.VMEM((ROWS_PER_TILE // L, L), jnp.float32),
        pltpu.VMEM((L,), jnp.int32),
        pltpu.VMEM((3, CHUNK, D), jnp.float32),
        pltpu.VMEM((CHUNK // 2, D), jnp.float32),
        pltpu.SemaphoreType.DMA,
        pltpu.SemaphoreType.DMA,
        pltpu.SemaphoreType.DMA,
        pltpu.SemaphoreType.DMA,
        pltpu.SemaphoreType.DMA,
        pltpu.SemaphoreType.DMA,
        pltpu.SemaphoreType.DMA,
    ],
)
def _dispatch_kernel(flow, src, gate, loads, zeros, out,
                     idx_v, gate_v, ld_v, bufs, zbuf,
                     sg0, sg1, sg2, ss0, ss1, ss2, sz):
    wid = _worker_id()
    e = wid // TILES_PER_EXPERT
    j = wid % TILES_PER_EXPERT
    vrows = ROWS_PER_TILE // L
    pltpu.sync_copy(src.at[pl.ds(wid * vrows, vrows)], idx_v)
    pltpu.sync_copy(gate.at[pl.ds(wid * vrows, vrows)], gate_v)
    pltpu.sync_copy(loads.at[e], ld_v)
    pltpu.sync_copy(zeros, zbuf)

    ld = jnp.max(ld_v[...])
    nc_expert = jnp.clip((ld + CHUNK - 1) // CHUNK, 0, CAPACITY // CHUNK)
    ngc = jnp.clip((nc_expert - j + TILES_PER_EXPERT - 1) // TILES_PER_EXPERT,
                   0, N_CHUNKS)

    sgs = (sg0, sg1, sg2)
    sss = (ss0, ss1, ss2)
    NB = 3
    ZR = CHUNK // 2
    obase = e * CAPACITY + j * CHUNK

    def out_rows(g):
        return out.at[pl.ds(obase + g * (TILES_PER_EXPERT * CHUNK), CHUNK)]

    def zrows(h):
        return out.at[pl.ds(obase + (h >> 1) * (TILES_PER_EXPERT * CHUNK)
                            + (h & 1) * ZR, ZR)]

    def gather_issue(g):
        @pl.when(g < ngc)
        def _():
            pltpu.async_copy(flow.at[idx_v.at[g]], bufs.at[g % NB], sgs[g % NB])

    def gather_wait(g):
        pltpu.make_async_copy(flow.at[idx_v.at[g]], bufs.at[g % NB], sgs[g % NB]).wait()

    def store_wait(g):
        @pl.when(g < ngc)
        def _():
            pltpu.make_async_copy(bufs.at[g % NB], out_rows(g), sss[g % NB]).wait()

    gather_issue(0)
    gather_issue(1)
    for g in range(N_CHUNKS):
        b = g % NB
        if g + 2 < N_CHUNKS:
            if g >= 1:
                store_wait(g - 1)
            gather_issue(g + 2)

        @pl.when(g < ngc)
        def _():
            gather_wait(g)

            def row_body(r, _):
                gs = plsc.load_gather(gate_v, [_splat_i32(g), _splat_i32(r)])

                def col_body(d, _):
                    bufs[b, r, pl.ds(d * L, L)] = bufs[b, r, pl.ds(d * L, L)] * gs
                    return 0

                lax.fori_loop(0, D // L, col_body, 0, unroll=8)
                return 0

            lax.fori_loop(0, CHUNK, row_body, 0)
            pltpu.async_copy(bufs.at[b], out_rows(g), sss[b])

        @pl.when(g >= ngc)
        def _():
            pltpu.async_copy(zbuf, zrows(2 * g), sz)
            pltpu.async_copy(zbuf, zrows(2 * g + 1), sz)

    store_wait(N_CHUNKS - 3)
    store_wait(N_CHUNKS - 2)
    store_wait(N_CHUNKS - 1)

    def zdrain(i, _):
        pltpu.make_async_copy(zbuf, zrows(0), sz).wait()
        return 0

    lax.fori_loop(2 * ngc, 2 * N_CHUNKS, zdrain, 0)


def kernel(in_flow, hot_mask, score):
    hmT = jnp.transpose(hot_mask.astype(jnp.int32))
    scoreT = jnp.transpose(score)
    src2d, gate2d, lidxT, loads2d = _index_kernel(hmT, scoreT)
    zeros = jnp.zeros((CHUNK // 2, D), jnp.float32)
    out = _dispatch_kernel(in_flow, src2d, gate2d, loads2d, zeros)
    return out, jnp.transpose(lidxT), loads2d[:, 0]
